# baseline (speedup 1.0000x reference)
import functools

import jax
import jax.numpy as jnp
from jax import lax
from jax.experimental import pallas as pl
from jax.experimental.pallas import tpu as pltpu
from jax.experimental.pallas import tpu_sc as plsc

NUM_CLASSES = 1_000_000
EMBED_DIM = 64
BATCH = 16384

NUM_CORES = 2
NUM_SUBCORES = 16
NUM_WORKERS = NUM_CORES * NUM_SUBCORES
B_PER_W = BATCH // NUM_WORKERS
HALF = B_PER_W // 2


@functools.partial(
    pl.kernel,
    out_type=jax.ShapeDtypeStruct((BATCH, EMBED_DIM), jnp.float32),
    mesh=plsc.VectorSubcoreMesh(core_axis_name="c", subcore_axis_name="s"),
    scratch_types=[
        pltpu.VMEM((B_PER_W,), jnp.int32),
        pltpu.VMEM((HALF, EMBED_DIM), jnp.float32),
        pltpu.SemaphoreType.DMA,
        pltpu.SemaphoreType.DMA,
    ],
)
def _gather_kernel(labels_hbm, table_hbm, out_hbm, idx_v, rows_v,
                   sem_st, sem_dma):
    wid = lax.axis_index("s") * NUM_CORES + lax.axis_index("c")
    base = wid * B_PER_W
    pltpu.sync_copy(labels_hbm.at[pl.ds(base, B_PER_W)], idx_v)

    @pl.loop(0, HALF // 16)
    def _issue(g):
        p0 = g * 16
        labs = idx_v[pl.ds(p0, 16)]
        labs2 = idx_v[pl.ds(HALF + p0, 16)]
        for i in range(16):
            pltpu.async_copy(table_hbm.at[labs[i]], rows_v.at[p0 + i], sem_st)
            pltpu.async_copy(table_hbm.at[labs2[i]],
                             out_hbm.at[base + HALF + p0 + i], sem_dma)

    pltpu.make_async_copy(table_hbm.at[pl.ds(0, HALF)], rows_v, sem_st).wait()
    pltpu.sync_copy(rows_v, out_hbm.at[pl.ds(base, HALF)])
    pltpu.make_async_copy(
        table_hbm.at[pl.ds(0, HALF)],
        out_hbm.at[pl.ds(base + HALF, HALF)],
        sem_dma,
    ).wait()


def kernel(labels, table):
    return _gather_kernel(labels.astype(jnp.int32), table)


# speedup vs baseline: 1.2829x; 1.2829x over previous
import functools

import jax
import jax.numpy as jnp
from jax import lax
from jax.experimental import pallas as pl
from jax.experimental.pallas import tpu as pltpu
from jax.experimental.pallas import tpu_sc as plsc

NUM_CLASSES = 1_000_000
EMBED_DIM = 64
BATCH = 16384

NUM_CORES = 2
NUM_SUBCORES = 16
NUM_WORKERS = NUM_CORES * NUM_SUBCORES
B_PER_W = BATCH // NUM_WORKERS
HALF = B_PER_W // 2


@functools.partial(
    pl.kernel,
    out_type=jax.ShapeDtypeStruct((BATCH, EMBED_DIM), jnp.float32),
    mesh=plsc.VectorSubcoreMesh(core_axis_name="c", subcore_axis_name="s"),
    scratch_types=[
        pltpu.VMEM((B_PER_W,), jnp.int32),
        pltpu.VMEM((HALF, EMBED_DIM), jnp.float32),
        pltpu.VMEM_SHARED((NUM_SUBCORES, HALF, EMBED_DIM), jnp.float32),
        pltpu.SemaphoreType.DMA,
        pltpu.SemaphoreType.DMA,
    ],
)
def _gather_kernel(labels_hbm, table_hbm, out_hbm, idx_v, rows_v, sp_v,
                   sem_st, sem_dma):
    sid = lax.axis_index("s")
    wid = sid * NUM_CORES + lax.axis_index("c")
    base = wid * B_PER_W
    pltpu.sync_copy(labels_hbm.at[pl.ds(base, B_PER_W)], idx_v)

    @pl.loop(0, HALF // 16)
    def _issue(g):
        p0 = g * 16
        labs = idx_v[pl.ds(p0, 16)]
        labs2 = idx_v[pl.ds(HALF + p0, 16)]
        for i in range(16):
            pltpu.async_copy(table_hbm.at[labs[i]], rows_v.at[p0 + i], sem_st)
            pltpu.async_copy(table_hbm.at[labs2[i]],
                             sp_v.at[sid, p0 + i], sem_dma)

    pltpu.make_async_copy(table_hbm.at[pl.ds(0, HALF)], rows_v, sem_st).wait()
    pltpu.sync_copy(rows_v, out_hbm.at[pl.ds(base, HALF)])
    pltpu.make_async_copy(table_hbm.at[pl.ds(0, HALF)], sp_v.at[sid],
                          sem_dma).wait()
    pltpu.sync_copy(sp_v.at[sid], out_hbm.at[pl.ds(base + HALF, HALF)])


def kernel(labels, table):
    return _gather_kernel(labels.astype(jnp.int32), table)
